# baseline (speedup 1.0000x reference)
import functools

import jax
import jax.numpy as jnp
from jax import lax
from jax.experimental import pallas as pl
from jax.experimental.pallas import tpu as pltpu
from jax.experimental.pallas import tpu_sc as plsc

N = 10000
E = 160000
D_EDGE = 16
HIDDEN = 128
LAYERS = 4
NUM_VOCAB = 5000
MAX_SEQ_LEN = 5
NUM_GRAPHS = 128

NC = 2
NS = 16
NW = NC * NS

K = 64
EB = 5120
EPAD = NW * EB
NBLK = EB // K
NACC = 10240
ROWS_PER_TILE = NACC // NS


def _sc_layer_kernel(h_hbm, e_hbm, src_hbm, dst_hbm, out_hbm,
                     sidx, didx, gbuf0, gbuf1, ebuf0, ebuf1, acc,
                     gsem0, gsem1, esem0, esem1, ssem0, ssem1):
    cid = lax.axis_index("c")
    sid = lax.axis_index("s")
    wid = sid * NC + cid
    gbuf = (gbuf0, gbuf1)
    ebuf = (ebuf0, ebuf1)
    gsem = (gsem0, gsem1)
    esem = (esem0, esem1)
    ssem = (ssem0, ssem1)

    pltpu.sync_copy(src_hbm.at[wid], sidx)
    pltpu.sync_copy(dst_hbm.at[wid], didx)

    def zrow(i, _):
        for j in range(HIDDEN // 16):
            gbuf0[i, pl.ds(j * 16, 16)] = jnp.zeros((16,), jnp.float32)
        return 0
    lax.fori_loop(0, K, zrow, 0)
    for r in range(ROWS_PER_TILE // K):
        pltpu.sync_copy(gbuf0, acc.at[pl.ds(sid * ROWS_PER_TILE + r * K, K)])
    plsc.subcore_barrier()

    def issue(q, p):
        base = pl.multiple_of(wid * EB + q * K, K)
        base2 = pl.multiple_of((wid * EB + q * K) // 2, K // 2)
        pltpu.async_copy(h_hbm.at[sidx.at[q]], gbuf[p], gsem[p])
        pltpu.async_copy(e_hbm.at[pl.ds(base2, K // 2)], ebuf[p], esem[p])

    def compute(p):
        g, e = gbuf[p], ebuf[p]
        himask = jnp.full((16,), -65536, jnp.int32)

        def pairrow(i2):
            for half in range(2):
                i = 2 * i2 + half
                for c in range(HIDDEN // 32):
                    ew = e[i2, pl.ds(64 * half + 16 * c, 16)]
                    e0 = lax.bitcast_convert_type(ew << 16, jnp.float32)
                    e1 = lax.bitcast_convert_type(ew & himask, jnp.float32)
                    s0 = pl.ds(32 * c, 16)
                    s1 = pl.ds(32 * c + 16, 16)
                    g[i, s0] = jnp.maximum(g[i, s0] + e0, 0.0)
                    g[i, s1] = jnp.maximum(g[i, s1] + e1, 0.0)
        plsc.parallel_loop(0, K // 2, unroll=2)(pairrow)

    def step(q, p):
        o = 1 - p
        pltpu.make_async_copy(h_hbm.at[sidx.at[q]], gbuf[p], gsem[p]).wait()
        pltpu.make_async_copy(e_hbm.at[pl.ds(0, K // 2)], ebuf[p],
                              esem[p]).wait()

        @pl.when(q >= 1)
        def _():
            pltpu.make_async_copy(gbuf[o], acc.at[didx.at[q]], ssem[o]).wait()

        @pl.when(q + 1 < NBLK)
        def _():
            issue(q + 1, o)

        compute(p)
        pltpu.async_copy(gbuf[p], acc.at[didx.at[q]], ssem[p], add=True)

    issue(0, 0)

    def pair(t, _):
        step(2 * t, 0)
        step(2 * t + 1, 1)
        return 0
    lax.fori_loop(0, NBLK // 2, pair, 0)
    pltpu.make_async_copy(gbuf[1], acc.at[didx.at[0]], ssem[1]).wait()
    plsc.subcore_barrier()
    pltpu.sync_copy(acc.at[pl.ds(sid * ROWS_PER_TILE, ROWS_PER_TILE)],
                    out_hbm.at[cid, pl.ds(sid * ROWS_PER_TILE, ROWS_PER_TILE)])


def _sc_layer(h, e_pad, src_p, dst_p):
    mesh = plsc.VectorSubcoreMesh(core_axis_name="c", subcore_axis_name="s")
    f = pl.kernel(
        _sc_layer_kernel,
        out_type=jax.ShapeDtypeStruct((NC, NACC, HIDDEN), jnp.float32),
        mesh=mesh,
        scratch_types=[
            pltpu.VMEM((NBLK, K), jnp.int32),
            pltpu.VMEM((NBLK, K), jnp.int32),
            pltpu.VMEM((K, HIDDEN), jnp.float32),
            pltpu.VMEM((K, HIDDEN), jnp.float32),
            pltpu.VMEM((K // 2, HIDDEN), jnp.int32),
            pltpu.VMEM((K // 2, HIDDEN), jnp.int32),
            pltpu.VMEM_SHARED((NACC, HIDDEN), jnp.float32),
            pltpu.SemaphoreType.DMA,
            pltpu.SemaphoreType.DMA,
            pltpu.SemaphoreType.DMA,
            pltpu.SemaphoreType.DMA,
            pltpu.SemaphoreType.DMA,
            pltpu.SemaphoreType.DMA,
        ],
    )
    return f(h, e_pad, src_p.reshape(NW, NBLK, K), dst_p.reshape(NW, NBLK, K))


def _encoder_body(x0_ref, x1_ref, d_ref, tt_ref, at_ref, dt_ref, out_ref):
    bm = x0_ref.shape[2]
    i0 = jax.lax.broadcasted_iota(jnp.int32, (bm, 128), 1)
    oh0 = (x0_ref[0, 0][:, None] == i0).astype(jnp.float32)
    oh1 = (x1_ref[0, 0][:, None] == i0).astype(jnp.float32)
    i2 = jax.lax.broadcasted_iota(jnp.int32, (bm, 32), 1)
    ohd = (d_ref[0, 0][:, None] == i2).astype(jnp.float32)
    out_ref[...] = (
        jnp.dot(oh0, tt_ref[...], preferred_element_type=jnp.float32)
        + jnp.dot(oh1, at_ref[...], preferred_element_type=jnp.float32)
        + jnp.dot(ohd, dt_ref[...], preferred_element_type=jnp.float32)
    )


def _encoder(x, node_depth, type_table, attr_table, depth_table):
    BM = 1000
    G = N // BM
    x0 = x[:, 0].astype(jnp.int32).reshape(G, 1, BM)
    x1 = x[:, 1].astype(jnp.int32).reshape(G, 1, BM)
    nd = node_depth.astype(jnp.int32).reshape(G, 1, BM)
    tt = jnp.zeros((128, HIDDEN), jnp.float32).at[:type_table.shape[0]].set(type_table)
    at = jnp.zeros((128, HIDDEN), jnp.float32).at[:attr_table.shape[0]].set(attr_table)
    dt = jnp.zeros((32, HIDDEN), jnp.float32).at[:depth_table.shape[0]].set(depth_table)
    return pl.pallas_call(
        _encoder_body,
        grid=(G,),
        in_specs=[
            pl.BlockSpec((1, 1, BM), lambda i: (i, 0, 0)),
            pl.BlockSpec((1, 1, BM), lambda i: (i, 0, 0)),
            pl.BlockSpec((1, 1, BM), lambda i: (i, 0, 0)),
            pl.BlockSpec((128, HIDDEN), lambda i: (0, 0)),
            pl.BlockSpec((128, HIDDEN), lambda i: (0, 0)),
            pl.BlockSpec((32, HIDDEN), lambda i: (0, 0)),
        ],
        out_specs=pl.BlockSpec((BM, HIDDEN), lambda i: (i, 0)),
        out_shape=jax.ShapeDtypeStruct((N, HIDDEN), jnp.float32),
    )(x0, x1, nd, tt, at, dt)


def _edge_encode_body(attr_ref, w_ref, b_ref, out_ref):
    bm = attr_ref.shape[0]
    row0 = pl.program_id(0) * bm
    rows = jax.lax.broadcasted_iota(jnp.int32, (bm, HIDDEN), 0) + row0
    val = (
        jnp.dot(attr_ref[...], w_ref[...], preferred_element_type=jnp.float32)
        + b_ref[...]
    )
    out_ref[...] = jnp.where(rows >= E, -1e30, val).astype(jnp.bfloat16)


def _edge_encode(edge_attr_pad, w, b):
    BM = 2048
    return pl.pallas_call(
        _edge_encode_body,
        grid=(EPAD // BM,),
        in_specs=[
            pl.BlockSpec((BM, D_EDGE), lambda i: (i, 0)),
            pl.BlockSpec((D_EDGE, HIDDEN), lambda i: (0, 0)),
            pl.BlockSpec((1, HIDDEN), lambda i: (0, 0)),
        ],
        out_specs=pl.BlockSpec((BM, HIDDEN), lambda i: (i, 0)),
        out_shape=jax.ShapeDtypeStruct((EPAD, HIDDEN), jnp.bfloat16),
    )(edge_attr_pad, w, b.reshape(1, HIDDEN))


def _permute_for_unpack(w):
    shape = w.shape
    return (w.reshape(shape[:-1] + (2, 2, 2, 16))
            .swapaxes(-1, -2)
            .reshape(shape))


def _mlp_body(scale_ref, h_ref, a0_ref, a1_ref, w1_ref, b1_ref, w2_ref, b2_ref,
              out_ref):
    z = scale_ref[0, 0] * h_ref[...] + a0_ref[...] + a1_ref[...]
    z = jnp.maximum(
        jnp.dot(z, w1_ref[...], preferred_element_type=jnp.float32) + b1_ref[...],
        0.0,
    )
    out_ref[...] = (
        jnp.dot(z, w2_ref[...], preferred_element_type=jnp.float32) + b2_ref[...]
    )


def _mlp(scale, h, a0, a1, w1, b1, w2, b2):
    BM = 1000
    return pl.pallas_call(
        _mlp_body,
        grid=(N // BM,),
        in_specs=[
            pl.BlockSpec(memory_space=pltpu.MemorySpace.SMEM),
            pl.BlockSpec((BM, HIDDEN), lambda i: (i, 0)),
            pl.BlockSpec((BM, HIDDEN), lambda i: (i, 0)),
            pl.BlockSpec((BM, HIDDEN), lambda i: (i, 0)),
            pl.BlockSpec((HIDDEN, 2 * HIDDEN), lambda i: (0, 0)),
            pl.BlockSpec((1, 2 * HIDDEN), lambda i: (0, 0)),
            pl.BlockSpec((2 * HIDDEN, HIDDEN), lambda i: (0, 0)),
            pl.BlockSpec((1, HIDDEN), lambda i: (0, 0)),
        ],
        out_specs=pl.BlockSpec((BM, HIDDEN), lambda i: (i, 0)),
        out_shape=jax.ShapeDtypeStruct((N, HIDDEN), jnp.float32),
    )(scale.reshape(1, 1), h, a0, a1, w1, b1.reshape(1, -1), w2,
      b2.reshape(1, -1))


def _pool_body(b_ref, h_ref, sum_ref, cnt_ref):
    i = pl.program_id(0)

    @pl.when(i == 0)
    def _():
        sum_ref[...] = jnp.zeros_like(sum_ref)
        cnt_ref[...] = jnp.zeros_like(cnt_ref)

    bm = b_ref.shape[2]
    ig = jax.lax.broadcasted_iota(jnp.int32, (bm, NUM_GRAPHS), 1)
    oh = (b_ref[0, 0][:, None] == ig).astype(jnp.float32)
    sum_ref[...] += lax.dot_general(
        oh, h_ref[...], (((0,), (0,)), ((), ())),
        preferred_element_type=jnp.float32)
    cnt_ref[...] += jnp.sum(oh, axis=0, keepdims=True)


def _pool(batch, h):
    BM = 1000
    G = N // BM
    b3 = batch.astype(jnp.int32).reshape(G, 1, BM)
    return pl.pallas_call(
        _pool_body,
        grid=(G,),
        in_specs=[
            pl.BlockSpec((1, 1, BM), lambda i: (i, 0, 0)),
            pl.BlockSpec((BM, HIDDEN), lambda i: (i, 0)),
        ],
        out_specs=[
            pl.BlockSpec((NUM_GRAPHS, HIDDEN), lambda i: (0, 0)),
            pl.BlockSpec((1, NUM_GRAPHS), lambda i: (0, 0)),
        ],
        out_shape=[
            jax.ShapeDtypeStruct((NUM_GRAPHS, HIDDEN), jnp.float32),
            jax.ShapeDtypeStruct((1, NUM_GRAPHS), jnp.float32),
        ],
    )(b3, h)


def _heads_body(h_ref, w_ref, b_ref, out_ref):
    out_ref[0] = (
        jnp.dot(h_ref[...], w_ref[0], preferred_element_type=jnp.float32)
        + b_ref[0]
    )


def _heads(h_graph, pred_w, pred_b):
    VPAD = 5120
    wp = jnp.zeros((MAX_SEQ_LEN, HIDDEN, VPAD), jnp.float32).at[:, :, :NUM_VOCAB].set(pred_w)
    bp = jnp.zeros((MAX_SEQ_LEN, 1, VPAD), jnp.float32).at[:, 0, :NUM_VOCAB].set(pred_b)
    out = pl.pallas_call(
        _heads_body,
        grid=(MAX_SEQ_LEN, 4),
        in_specs=[
            pl.BlockSpec((NUM_GRAPHS, HIDDEN), lambda i, j: (0, 0)),
            pl.BlockSpec((1, HIDDEN, VPAD // 4), lambda i, j: (i, 0, j)),
            pl.BlockSpec((1, 1, VPAD // 4), lambda i, j: (i, 0, j)),
        ],
        out_specs=pl.BlockSpec((1, NUM_GRAPHS, VPAD // 4), lambda i, j: (i, 0, j)),
        out_shape=jax.ShapeDtypeStruct((MAX_SEQ_LEN, NUM_GRAPHS, VPAD), jnp.float32),
    )(h_graph, wp, bp)
    return out[:, :, :NUM_VOCAB]


def kernel(x, edge_index, edge_attr, node_depth, batch, type_table, attr_table,
           depth_table, eps, edge_W, edge_b, W1, b1, W2, b2, pred_W, pred_b):
    npad = EPAD - E
    pad_iota = lax.iota(jnp.int32, npad)
    src_p = jnp.concatenate([edge_index[0].astype(jnp.int32), pad_iota % N])
    dst_p = jnp.concatenate(
        [edge_index[1].astype(jnp.int32), (pad_iota * 37) % N])
    attr_p = jnp.concatenate(
        [edge_attr, jnp.zeros((npad, D_EDGE), jnp.float32)])
    edge_Wp = _permute_for_unpack(edge_W)
    edge_bp = _permute_for_unpack(edge_b)

    h = _encoder(x, node_depth, type_table, attr_table, depth_table)
    for i in range(LAYERS):
        e = _edge_encode(attr_p, edge_Wp[i], edge_bp[i])
        ei = lax.bitcast_convert_type(
            e.reshape(EPAD, HIDDEN // 2, 2), jnp.int32).reshape(
                EPAD // 2, HIDDEN)
        agg = _sc_layer(h, ei, src_p, dst_p)
        h = _mlp(1.0 + eps[i], h, agg[0, :N], agg[1, :N],
                 W1[i], b1[i], W2[i], b2[i])
    sums, counts = _pool(batch, h)
    h_graph = sums / jnp.maximum(counts.reshape(NUM_GRAPHS, 1), 1.0)
    return _heads(h_graph, pred_W, pred_b)


# speedup vs baseline: 2.9383x; 2.9383x over previous
import functools

import jax
import jax.numpy as jnp
from jax import lax
from jax.experimental import pallas as pl
from jax.experimental.pallas import tpu as pltpu
from jax.experimental.pallas import tpu_sc as plsc

N = 10000
E = 160000
D_EDGE = 16
HIDDEN = 128
LAYERS = 4
NUM_VOCAB = 5000
MAX_SEQ_LEN = 5
NUM_GRAPHS = 128

NC = 2
NS = 16
NW = NC * NS

K = 64
EB = 5120
EPAD = NW * EB
NBLK = EB // K
NACC = 10240
ROWS_PER_TILE = NACC // NS


def _sc_layer_kernel(h_hbm, e_hbm, src_hbm, dst_hbm, out_hbm,
                     sidx, didx, gbuf0, gbuf1, ebuf0, ebuf1, acc,
                     gsem0, gsem1, esem0, esem1, ssem0, ssem1):
    cid = lax.axis_index("c")
    sid = lax.axis_index("s")
    wid = sid * NC + cid
    gbuf = (gbuf0, gbuf1)
    ebuf = (ebuf0, ebuf1)
    gsem = (gsem0, gsem1)
    esem = (esem0, esem1)
    ssem = (ssem0, ssem1)

    pltpu.sync_copy(src_hbm.at[wid], sidx)
    pltpu.sync_copy(dst_hbm.at[wid], didx)

    def zrow(i, _):
        for j in range(HIDDEN // 16):
            gbuf0[i, pl.ds(j * 16, 16)] = jnp.zeros((16,), jnp.float32)
        return 0
    lax.fori_loop(0, K, zrow, 0)
    for r in range(ROWS_PER_TILE // K):
        pltpu.sync_copy(gbuf0, acc.at[pl.ds(sid * ROWS_PER_TILE + r * K, K)])
    plsc.subcore_barrier()

    def sidx_slice(q):
        return sidx.at[q >> 1, pl.ds(pl.multiple_of((q & 1) * K, K), K)]

    def issue(q, p):
        base = pl.multiple_of(wid * EB + q * K, K)
        pltpu.async_copy(h_hbm.at[sidx_slice(q)], gbuf[p], gsem[p])
        pltpu.async_copy(e_hbm.at[pl.ds(base, K)], ebuf[p], esem[p])

    def compute(p):
        g, e = gbuf[p], ebuf[p]
        himask = jnp.full((16,), -65536, jnp.int32)

        def row(i):
            for c in range(HIDDEN // 32):
                ew = e[i, pl.ds(16 * c, 16)]
                e0 = lax.bitcast_convert_type(ew << 16, jnp.float32)
                e1 = lax.bitcast_convert_type(ew & himask, jnp.float32)
                s0 = pl.ds(32 * c, 16)
                s1 = pl.ds(32 * c + 16, 16)
                g[i, s0] = jnp.maximum(g[i, s0] + e0, 0.0)
                g[i, s1] = jnp.maximum(g[i, s1] + e1, 0.0)
        plsc.parallel_loop(0, K, unroll=4)(row)

    def step(q, p):
        o = 1 - p
        pltpu.make_async_copy(h_hbm.at[sidx_slice(q)], gbuf[p], gsem[p]).wait()
        pltpu.make_async_copy(e_hbm.at[pl.ds(0, K)], ebuf[p], esem[p]).wait()

        @pl.when(q >= 1)
        def _():
            pltpu.make_async_copy(gbuf[o], acc.at[didx.at[q]], ssem[o]).wait()

        @pl.when(q + 1 < NBLK)
        def _():
            issue(q + 1, o)

        compute(p)
        pltpu.async_copy(gbuf[p], acc.at[didx.at[q]], ssem[p], add=True)

    issue(0, 0)

    def pair(t, _):
        step(2 * t, 0)
        step(2 * t + 1, 1)
        return 0
    lax.fori_loop(0, NBLK // 2, pair, 0)
    pltpu.make_async_copy(gbuf[1], acc.at[didx.at[0]], ssem[1]).wait()
    plsc.subcore_barrier()
    pltpu.sync_copy(acc.at[pl.ds(sid * ROWS_PER_TILE, ROWS_PER_TILE)],
                    out_hbm.at[cid, pl.ds(sid * ROWS_PER_TILE, ROWS_PER_TILE)])


def _sc_layer(h, e_pad, src_p, dst_p):
    mesh = plsc.VectorSubcoreMesh(core_axis_name="c", subcore_axis_name="s")
    f = pl.kernel(
        _sc_layer_kernel,
        out_type=jax.ShapeDtypeStruct((NC, NACC, HIDDEN), jnp.float32),
        mesh=mesh,
        scratch_types=[
            pltpu.VMEM((NBLK // 2, 2 * K), jnp.int32),
            pltpu.VMEM((NBLK, K), jnp.int32),
            pltpu.VMEM((K, HIDDEN), jnp.float32),
            pltpu.VMEM((K, HIDDEN), jnp.float32),
            pltpu.VMEM((K, HIDDEN // 2), jnp.int32),
            pltpu.VMEM((K, HIDDEN // 2), jnp.int32),
            pltpu.VMEM_SHARED((NACC, HIDDEN), jnp.float32),
            pltpu.SemaphoreType.DMA,
            pltpu.SemaphoreType.DMA,
            pltpu.SemaphoreType.DMA,
            pltpu.SemaphoreType.DMA,
            pltpu.SemaphoreType.DMA,
            pltpu.SemaphoreType.DMA,
        ],
    )
    return f(h, e_pad, src_p.reshape(NW, NBLK // 2, 2 * K),
             dst_p.reshape(NW, NBLK, K))


def _encoder_body(x0_ref, x1_ref, d_ref, tt_ref, at_ref, dt_ref, out_ref):
    bm = x0_ref.shape[2]
    i0 = jax.lax.broadcasted_iota(jnp.int32, (bm, 128), 1)
    oh0 = (x0_ref[0, 0][:, None] == i0).astype(jnp.float32)
    oh1 = (x1_ref[0, 0][:, None] == i0).astype(jnp.float32)
    i2 = jax.lax.broadcasted_iota(jnp.int32, (bm, 32), 1)
    ohd = (d_ref[0, 0][:, None] == i2).astype(jnp.float32)
    out_ref[...] = (
        jnp.dot(oh0, tt_ref[...], preferred_element_type=jnp.float32)
        + jnp.dot(oh1, at_ref[...], preferred_element_type=jnp.float32)
        + jnp.dot(ohd, dt_ref[...], preferred_element_type=jnp.float32)
    )


def _encoder(x, node_depth, type_table, attr_table, depth_table):
    BM = 1000
    G = N // BM
    x0 = x[:, 0].astype(jnp.int32).reshape(G, 1, BM)
    x1 = x[:, 1].astype(jnp.int32).reshape(G, 1, BM)
    nd = node_depth.astype(jnp.int32).reshape(G, 1, BM)
    tt = jnp.zeros((128, HIDDEN), jnp.float32).at[:type_table.shape[0]].set(type_table)
    at = jnp.zeros((128, HIDDEN), jnp.float32).at[:attr_table.shape[0]].set(attr_table)
    dt = jnp.zeros((32, HIDDEN), jnp.float32).at[:depth_table.shape[0]].set(depth_table)
    return pl.pallas_call(
        _encoder_body,
        grid=(G,),
        in_specs=[
            pl.BlockSpec((1, 1, BM), lambda i: (i, 0, 0)),
            pl.BlockSpec((1, 1, BM), lambda i: (i, 0, 0)),
            pl.BlockSpec((1, 1, BM), lambda i: (i, 0, 0)),
            pl.BlockSpec((128, HIDDEN), lambda i: (0, 0)),
            pl.BlockSpec((128, HIDDEN), lambda i: (0, 0)),
            pl.BlockSpec((32, HIDDEN), lambda i: (0, 0)),
        ],
        out_specs=pl.BlockSpec((BM, HIDDEN), lambda i: (i, 0)),
        out_shape=jax.ShapeDtypeStruct((N, HIDDEN), jnp.float32),
    )(x0, x1, nd, tt, at, dt)


def _rne_bf16_bits(v):
    u = lax.bitcast_convert_type(v, jnp.int32)
    return u + 0x7FFF + ((u >> 16) & 1)


def _edge_encode_body(attr_ref, wx_ref, bx_ref, wy_ref, by_ref, out_ref):
    bm = attr_ref.shape[0]
    row0 = pl.program_id(0) * bm
    rows = jax.lax.broadcasted_iota(jnp.int32, (bm, HIDDEN // 2), 0) + row0
    vx = (
        jnp.dot(attr_ref[...], wx_ref[...], preferred_element_type=jnp.float32)
        + bx_ref[...]
    )
    vy = (
        jnp.dot(attr_ref[...], wy_ref[...], preferred_element_type=jnp.float32)
        + by_ref[...]
    )
    pad = rows >= E
    ux = _rne_bf16_bits(jnp.where(pad, -1e30, vx))
    uy = _rne_bf16_bits(jnp.where(pad, -1e30, vy))
    out_ref[...] = ((ux >> 16) & 0xFFFF) | (uy & jnp.int32(-65536))


def _edge_encode(edge_attr_pad, wx, bx, wy, by):
    BM = 2048
    return pl.pallas_call(
        _edge_encode_body,
        grid=(EPAD // BM,),
        in_specs=[
            pl.BlockSpec((BM, D_EDGE), lambda i: (i, 0)),
            pl.BlockSpec((D_EDGE, HIDDEN // 2), lambda i: (0, 0)),
            pl.BlockSpec((1, HIDDEN // 2), lambda i: (0, 0)),
            pl.BlockSpec((D_EDGE, HIDDEN // 2), lambda i: (0, 0)),
            pl.BlockSpec((1, HIDDEN // 2), lambda i: (0, 0)),
        ],
        out_specs=pl.BlockSpec((BM, HIDDEN // 2), lambda i: (i, 0)),
        out_shape=jax.ShapeDtypeStruct((EPAD, HIDDEN // 2), jnp.int32),
    )(edge_attr_pad, wx, bx.reshape(1, -1), wy, by.reshape(1, -1))


def _mlp_body(scale_ref, h_ref, a0_ref, a1_ref, w1_ref, b1_ref, w2_ref, b2_ref,
              out_ref):
    z = scale_ref[0, 0] * h_ref[...] + a0_ref[...] + a1_ref[...]
    z = jnp.maximum(
        jnp.dot(z, w1_ref[...], preferred_element_type=jnp.float32) + b1_ref[...],
        0.0,
    )
    out_ref[...] = (
        jnp.dot(z, w2_ref[...], preferred_element_type=jnp.float32) + b2_ref[...]
    )


def _mlp(scale, h, a0, a1, w1, b1, w2, b2):
    BM = 1000
    return pl.pallas_call(
        _mlp_body,
        grid=(N // BM,),
        in_specs=[
            pl.BlockSpec(memory_space=pltpu.MemorySpace.SMEM),
            pl.BlockSpec((BM, HIDDEN), lambda i: (i, 0)),
            pl.BlockSpec((BM, HIDDEN), lambda i: (i, 0)),
            pl.BlockSpec((BM, HIDDEN), lambda i: (i, 0)),
            pl.BlockSpec((HIDDEN, 2 * HIDDEN), lambda i: (0, 0)),
            pl.BlockSpec((1, 2 * HIDDEN), lambda i: (0, 0)),
            pl.BlockSpec((2 * HIDDEN, HIDDEN), lambda i: (0, 0)),
            pl.BlockSpec((1, HIDDEN), lambda i: (0, 0)),
        ],
        out_specs=pl.BlockSpec((BM, HIDDEN), lambda i: (i, 0)),
        out_shape=jax.ShapeDtypeStruct((N, HIDDEN), jnp.float32),
    )(scale.reshape(1, 1), h, a0, a1, w1, b1.reshape(1, -1), w2,
      b2.reshape(1, -1))


def _pool_body(b_ref, h_ref, sum_ref, cnt_ref):
    i = pl.program_id(0)

    @pl.when(i == 0)
    def _():
        sum_ref[...] = jnp.zeros_like(sum_ref)
        cnt_ref[...] = jnp.zeros_like(cnt_ref)

    bm = b_ref.shape[2]
    ig = jax.lax.broadcasted_iota(jnp.int32, (bm, NUM_GRAPHS), 1)
    oh = (b_ref[0, 0][:, None] == ig).astype(jnp.float32)
    sum_ref[...] += lax.dot_general(
        oh, h_ref[...], (((0,), (0,)), ((), ())),
        preferred_element_type=jnp.float32)
    cnt_ref[...] += jnp.sum(oh, axis=0, keepdims=True)


def _pool(batch, h):
    BM = 1000
    G = N // BM
    b3 = batch.astype(jnp.int32).reshape(G, 1, BM)
    return pl.pallas_call(
        _pool_body,
        grid=(G,),
        in_specs=[
            pl.BlockSpec((1, 1, BM), lambda i: (i, 0, 0)),
            pl.BlockSpec((BM, HIDDEN), lambda i: (i, 0)),
        ],
        out_specs=[
            pl.BlockSpec((NUM_GRAPHS, HIDDEN), lambda i: (0, 0)),
            pl.BlockSpec((1, NUM_GRAPHS), lambda i: (0, 0)),
        ],
        out_shape=[
            jax.ShapeDtypeStruct((NUM_GRAPHS, HIDDEN), jnp.float32),
            jax.ShapeDtypeStruct((1, NUM_GRAPHS), jnp.float32),
        ],
    )(b3, h)


def _heads_body(h_ref, w_ref, b_ref, out_ref):
    out_ref[0] = (
        jnp.dot(h_ref[...], w_ref[0], preferred_element_type=jnp.float32)
        + b_ref[0]
    )


def _heads(h_graph, pred_w, pred_b):
    VPAD = 5120
    wp = jnp.zeros((MAX_SEQ_LEN, HIDDEN, VPAD), jnp.float32).at[:, :, :NUM_VOCAB].set(pred_w)
    bp = jnp.zeros((MAX_SEQ_LEN, 1, VPAD), jnp.float32).at[:, 0, :NUM_VOCAB].set(pred_b)
    out = pl.pallas_call(
        _heads_body,
        grid=(MAX_SEQ_LEN, 4),
        in_specs=[
            pl.BlockSpec((NUM_GRAPHS, HIDDEN), lambda i, j: (0, 0)),
            pl.BlockSpec((1, HIDDEN, VPAD // 4), lambda i, j: (i, 0, j)),
            pl.BlockSpec((1, 1, VPAD // 4), lambda i, j: (i, 0, j)),
        ],
        out_specs=pl.BlockSpec((1, NUM_GRAPHS, VPAD // 4), lambda i, j: (i, 0, j)),
        out_shape=jax.ShapeDtypeStruct((MAX_SEQ_LEN, NUM_GRAPHS, VPAD), jnp.float32),
    )(h_graph, wp, bp)
    return out[:, :, :NUM_VOCAB]


def kernel(x, edge_index, edge_attr, node_depth, batch, type_table, attr_table,
           depth_table, eps, edge_W, edge_b, W1, b1, W2, b2, pred_W, pred_b):
    npad = EPAD - E
    pad_iota = lax.iota(jnp.int32, npad)
    src_p = jnp.concatenate([edge_index[0].astype(jnp.int32), pad_iota % N])
    dst_p = jnp.concatenate(
        [edge_index[1].astype(jnp.int32), (pad_iota * 37) % N])
    attr_p = jnp.concatenate(
        [edge_attr, jnp.zeros((npad, D_EDGE), jnp.float32)])
    xcols = jnp.array([32 * (w // 16) + w % 16 for w in range(HIDDEN // 2)],
                      jnp.int32)
    wx, bx = edge_W[:, :, xcols], edge_b[:, xcols]
    wy, by = edge_W[:, :, xcols + 16], edge_b[:, xcols + 16]

    h = _encoder(x, node_depth, type_table, attr_table, depth_table)
    for i in range(LAYERS):
        ei = _edge_encode(attr_p, wx[i], bx[i], wy[i], by[i])
        agg = _sc_layer(h, ei, src_p, dst_p)
        h = _mlp(1.0 + eps[i], h, agg[0, :N], agg[1, :N],
                 W1[i], b1[i], W2[i], b2[i])
    sums, counts = _pool(batch, h)
    h_graph = sums / jnp.maximum(counts.reshape(NUM_GRAPHS, 1), 1.0)
    return _heads(h_graph, pred_W, pred_b)


# speedup vs baseline: 3.2972x; 1.1222x over previous
import functools

import jax
import jax.numpy as jnp
from jax import lax
from jax.experimental import pallas as pl
from jax.experimental.pallas import tpu as pltpu
from jax.experimental.pallas import tpu_sc as plsc

N = 10000
E = 160000
D_EDGE = 16
HIDDEN = 128
LAYERS = 4
NUM_VOCAB = 5000
MAX_SEQ_LEN = 5
NUM_GRAPHS = 128

NC = 2
NS = 16
NW = NC * NS

K = 64
EB = 5120
EPAD = NW * EB
NBLK = EB // K
NACC = 10240
ROWS_PER_TILE = NACC // NS


def _sc_layer_kernel(h_hbm, e_hbm, src_hbm, dst_hbm, out_hbm,
                     sidx, didx, gbuf0, gbuf1, ebuf0, ebuf1, acc,
                     gsem0, gsem1, esem0, esem1, ssem0, ssem1):
    cid = lax.axis_index("c")
    sid = lax.axis_index("s")
    wid = sid * NC + cid
    gbuf = (gbuf0, gbuf1)
    ebuf = (ebuf0, ebuf1)
    gsem = (gsem0, gsem1)
    esem = (esem0, esem1)
    ssem = (ssem0, ssem1)

    pltpu.sync_copy(src_hbm.at[wid], sidx)
    pltpu.sync_copy(dst_hbm.at[wid], didx)

    def zrow(i, _):
        for j in range(HIDDEN // 16):
            gbuf0[i, pl.ds(j * 16, 16)] = jnp.zeros((16,), jnp.float32)
        return 0
    lax.fori_loop(0, K, zrow, 0)
    for r in range(ROWS_PER_TILE // K):
        pltpu.sync_copy(gbuf0, acc.at[pl.ds(sid * ROWS_PER_TILE + r * K, K)])
    plsc.subcore_barrier()

    def sidx_slice(q):
        return sidx.at[q >> 1, pl.ds(pl.multiple_of((q & 1) * K, K), K)]

    def issue(q, p):
        base2 = pl.multiple_of((wid * EB + q * K) // 2, K // 2)
        pltpu.async_copy(h_hbm.at[sidx_slice(q)], gbuf[p], gsem[p])
        pltpu.async_copy(e_hbm.at[pl.ds(base2, K // 2)], ebuf[p], esem[p])

    def compute(p):
        g, e = gbuf[p], ebuf[p]
        himask = jnp.full((16,), -65536, jnp.int32)

        def pairrow(i2):
            for half in range(2):
                i = 2 * i2 + half
                for c in range(HIDDEN // 32):
                    ew = e[i2, pl.ds(64 * half + 16 * c, 16)]
                    e0 = lax.bitcast_convert_type(ew << 16, jnp.float32)
                    e1 = lax.bitcast_convert_type(ew & himask, jnp.float32)
                    s0 = pl.ds(32 * c, 16)
                    s1 = pl.ds(32 * c + 16, 16)
                    g[i, s0] = jnp.maximum(g[i, s0] + e0, 0.0)
                    g[i, s1] = jnp.maximum(g[i, s1] + e1, 0.0)
        plsc.parallel_loop(0, K // 2, unroll=2)(pairrow)

    def step(q, p):
        o = 1 - p
        pltpu.make_async_copy(h_hbm.at[sidx_slice(q)], gbuf[p], gsem[p]).wait()
        pltpu.make_async_copy(e_hbm.at[pl.ds(0, K // 2)], ebuf[p],
                              esem[p]).wait()

        @pl.when(q >= 1)
        def _():
            pltpu.make_async_copy(gbuf[o], acc.at[didx.at[q]], ssem[o]).wait()

        @pl.when(q + 1 < NBLK)
        def _():
            issue(q + 1, o)

        compute(p)
        pltpu.async_copy(gbuf[p], acc.at[didx.at[q]], ssem[p], add=True)

    issue(0, 0)

    def pair(t, _):
        step(2 * t, 0)
        step(2 * t + 1, 1)
        return 0
    lax.fori_loop(0, NBLK // 2, pair, 0)
    pltpu.make_async_copy(gbuf[1], acc.at[didx.at[0]], ssem[1]).wait()
    plsc.subcore_barrier()
    pltpu.sync_copy(acc.at[pl.ds(sid * ROWS_PER_TILE, ROWS_PER_TILE)],
                    out_hbm.at[cid, pl.ds(sid * ROWS_PER_TILE, ROWS_PER_TILE)])


def _sc_layer(h, e_pad, src_p, dst_p):
    mesh = plsc.VectorSubcoreMesh(core_axis_name="c", subcore_axis_name="s")
    f = pl.kernel(
        _sc_layer_kernel,
        out_type=jax.ShapeDtypeStruct((NC, NACC, HIDDEN), jnp.float32),
        mesh=mesh,
        scratch_types=[
            pltpu.VMEM((NBLK // 2, 2 * K), jnp.int32),
            pltpu.VMEM((NBLK, K), jnp.int32),
            pltpu.VMEM((K, HIDDEN), jnp.float32),
            pltpu.VMEM((K, HIDDEN), jnp.float32),
            pltpu.VMEM((K // 2, HIDDEN), jnp.int32),
            pltpu.VMEM((K // 2, HIDDEN), jnp.int32),
            pltpu.VMEM_SHARED((NACC, HIDDEN), jnp.float32),
            pltpu.SemaphoreType.DMA,
            pltpu.SemaphoreType.DMA,
            pltpu.SemaphoreType.DMA,
            pltpu.SemaphoreType.DMA,
            pltpu.SemaphoreType.DMA,
            pltpu.SemaphoreType.DMA,
        ],
    )
    return f(h, e_pad, src_p.reshape(NW, NBLK // 2, 2 * K),
             dst_p.reshape(NW, NBLK, K))


def _encoder_body(x0_ref, x1_ref, d_ref, tt_ref, at_ref, dt_ref, out_ref):
    bm = x0_ref.shape[2]
    i0 = jax.lax.broadcasted_iota(jnp.int32, (bm, 128), 1)
    oh0 = (x0_ref[0, 0][:, None] == i0).astype(jnp.float32)
    oh1 = (x1_ref[0, 0][:, None] == i0).astype(jnp.float32)
    i2 = jax.lax.broadcasted_iota(jnp.int32, (bm, 32), 1)
    ohd = (d_ref[0, 0][:, None] == i2).astype(jnp.float32)
    out_ref[...] = (
        jnp.dot(oh0, tt_ref[...], preferred_element_type=jnp.float32)
        + jnp.dot(oh1, at_ref[...], preferred_element_type=jnp.float32)
        + jnp.dot(ohd, dt_ref[...], preferred_element_type=jnp.float32)
    )


def _encoder(x, node_depth, type_table, attr_table, depth_table):
    BM = 1000
    G = N // BM
    x0 = x[:, 0].astype(jnp.int32).reshape(G, 1, BM)
    x1 = x[:, 1].astype(jnp.int32).reshape(G, 1, BM)
    nd = node_depth.astype(jnp.int32).reshape(G, 1, BM)
    tt = jnp.zeros((128, HIDDEN), jnp.float32).at[:type_table.shape[0]].set(type_table)
    at = jnp.zeros((128, HIDDEN), jnp.float32).at[:attr_table.shape[0]].set(attr_table)
    dt = jnp.zeros((32, HIDDEN), jnp.float32).at[:depth_table.shape[0]].set(depth_table)
    return pl.pallas_call(
        _encoder_body,
        grid=(G,),
        in_specs=[
            pl.BlockSpec((1, 1, BM), lambda i: (i, 0, 0)),
            pl.BlockSpec((1, 1, BM), lambda i: (i, 0, 0)),
            pl.BlockSpec((1, 1, BM), lambda i: (i, 0, 0)),
            pl.BlockSpec((128, HIDDEN), lambda i: (0, 0)),
            pl.BlockSpec((128, HIDDEN), lambda i: (0, 0)),
            pl.BlockSpec((32, HIDDEN), lambda i: (0, 0)),
        ],
        out_specs=pl.BlockSpec((BM, HIDDEN), lambda i: (i, 0)),
        out_shape=jax.ShapeDtypeStruct((N, HIDDEN), jnp.float32),
    )(x0, x1, nd, tt, at, dt)


def _rne_bf16_bits(v):
    u = lax.bitcast_convert_type(v, jnp.int32)
    return u + 0x7FFF + ((u >> 16) & 1)


def _edge_encode_body(attr_ref, wx_ref, bx_ref, wy_ref, by_ref, out_ref):
    bm = attr_ref.shape[0]
    row0 = pl.program_id(0) * bm
    pad = (jax.lax.broadcasted_iota(jnp.int32, (bm, HIDDEN // 2), 0) + row0
           ) >= E // 2

    def packed(a):
        vx = jnp.dot(a, wx_ref[...], preferred_element_type=jnp.float32) + bx_ref[...]
        vy = jnp.dot(a, wy_ref[...], preferred_element_type=jnp.float32) + by_ref[...]
        ux = _rne_bf16_bits(jnp.where(pad, -1e30, vx))
        uy = _rne_bf16_bits(jnp.where(pad, -1e30, vy))
        return ((ux >> 16) & 0xFFFF) | (uy & jnp.int32(-65536))

    out_ref[...] = jnp.concatenate(
        [packed(attr_ref[:, :D_EDGE]), packed(attr_ref[:, D_EDGE:])], axis=1)


def _edge_encode(attr2_pad, wx, bx, wy, by):
    BM = 2048
    return pl.pallas_call(
        _edge_encode_body,
        grid=(EPAD // 2 // BM,),
        in_specs=[
            pl.BlockSpec((BM, 2 * D_EDGE), lambda i: (i, 0)),
            pl.BlockSpec((D_EDGE, HIDDEN // 2), lambda i: (0, 0)),
            pl.BlockSpec((1, HIDDEN // 2), lambda i: (0, 0)),
            pl.BlockSpec((D_EDGE, HIDDEN // 2), lambda i: (0, 0)),
            pl.BlockSpec((1, HIDDEN // 2), lambda i: (0, 0)),
        ],
        out_specs=pl.BlockSpec((BM, HIDDEN), lambda i: (i, 0)),
        out_shape=jax.ShapeDtypeStruct((EPAD // 2, HIDDEN), jnp.int32),
    )(attr2_pad, wx, bx.reshape(1, -1), wy, by.reshape(1, -1))


def _mlp_body(scale_ref, h_ref, a0_ref, a1_ref, w1_ref, b1_ref, w2_ref, b2_ref,
              out_ref):
    z = scale_ref[0, 0] * h_ref[...] + a0_ref[0] + a1_ref[0]
    z = jnp.maximum(
        jnp.dot(z, w1_ref[...], preferred_element_type=jnp.float32) + b1_ref[...],
        0.0,
    )
    out_ref[...] = (
        jnp.dot(z, w2_ref[...], preferred_element_type=jnp.float32) + b2_ref[...]
    )


def _mlp(scale, h, agg, w1, b1, w2, b2):
    BM = 1000
    return pl.pallas_call(
        _mlp_body,
        grid=(N // BM,),
        in_specs=[
            pl.BlockSpec(memory_space=pltpu.MemorySpace.SMEM),
            pl.BlockSpec((BM, HIDDEN), lambda i: (i, 0)),
            pl.BlockSpec((1, BM, HIDDEN), lambda i: (0, i, 0)),
            pl.BlockSpec((1, BM, HIDDEN), lambda i: (1, i, 0)),
            pl.BlockSpec((HIDDEN, 2 * HIDDEN), lambda i: (0, 0)),
            pl.BlockSpec((1, 2 * HIDDEN), lambda i: (0, 0)),
            pl.BlockSpec((2 * HIDDEN, HIDDEN), lambda i: (0, 0)),
            pl.BlockSpec((1, HIDDEN), lambda i: (0, 0)),
        ],
        out_specs=pl.BlockSpec((BM, HIDDEN), lambda i: (i, 0)),
        out_shape=jax.ShapeDtypeStruct((N, HIDDEN), jnp.float32),
    )(scale.reshape(1, 1), h, agg, agg, w1, b1.reshape(1, -1), w2,
      b2.reshape(1, -1))


def _pool_body(b_ref, h_ref, sum_ref, cnt_ref):
    i = pl.program_id(0)

    @pl.when(i == 0)
    def _():
        sum_ref[...] = jnp.zeros_like(sum_ref)
        cnt_ref[...] = jnp.zeros_like(cnt_ref)

    bm = b_ref.shape[2]
    ig = jax.lax.broadcasted_iota(jnp.int32, (bm, NUM_GRAPHS), 1)
    oh = (b_ref[0, 0][:, None] == ig).astype(jnp.float32)
    sum_ref[...] += lax.dot_general(
        oh, h_ref[...], (((0,), (0,)), ((), ())),
        preferred_element_type=jnp.float32)
    cnt_ref[...] += jnp.sum(oh, axis=0, keepdims=True)


def _pool(batch, h):
    BM = 1000
    G = N // BM
    b3 = batch.astype(jnp.int32).reshape(G, 1, BM)
    return pl.pallas_call(
        _pool_body,
        grid=(G,),
        in_specs=[
            pl.BlockSpec((1, 1, BM), lambda i: (i, 0, 0)),
            pl.BlockSpec((BM, HIDDEN), lambda i: (i, 0)),
        ],
        out_specs=[
            pl.BlockSpec((NUM_GRAPHS, HIDDEN), lambda i: (0, 0)),
            pl.BlockSpec((1, NUM_GRAPHS), lambda i: (0, 0)),
        ],
        out_shape=[
            jax.ShapeDtypeStruct((NUM_GRAPHS, HIDDEN), jnp.float32),
            jax.ShapeDtypeStruct((1, NUM_GRAPHS), jnp.float32),
        ],
    )(b3, h)


def _heads_body(h_ref, w_ref, b_ref, out_ref):
    out_ref[0] = (
        jnp.dot(h_ref[...], w_ref[0], preferred_element_type=jnp.float32)
        + b_ref[0]
    )


def _heads(h_graph, pred_w, pred_b):
    VPAD = 5120
    wp = jnp.zeros((MAX_SEQ_LEN, HIDDEN, VPAD), jnp.float32).at[:, :, :NUM_VOCAB].set(pred_w)
    bp = jnp.zeros((MAX_SEQ_LEN, 1, VPAD), jnp.float32).at[:, 0, :NUM_VOCAB].set(pred_b)
    out = pl.pallas_call(
        _heads_body,
        grid=(MAX_SEQ_LEN, 4),
        in_specs=[
            pl.BlockSpec((NUM_GRAPHS, HIDDEN), lambda i, j: (0, 0)),
            pl.BlockSpec((1, HIDDEN, VPAD // 4), lambda i, j: (i, 0, j)),
            pl.BlockSpec((1, 1, VPAD // 4), lambda i, j: (i, 0, j)),
        ],
        out_specs=pl.BlockSpec((1, NUM_GRAPHS, VPAD // 4), lambda i, j: (i, 0, j)),
        out_shape=jax.ShapeDtypeStruct((MAX_SEQ_LEN, NUM_GRAPHS, VPAD), jnp.float32),
    )(h_graph, wp, bp)
    return out[:, :, :NUM_VOCAB]


def kernel(x, edge_index, edge_attr, node_depth, batch, type_table, attr_table,
           depth_table, eps, edge_W, edge_b, W1, b1, W2, b2, pred_W, pred_b):
    npad = EPAD - E
    pad_iota = lax.iota(jnp.int32, npad)
    src_p = jnp.concatenate([edge_index[0].astype(jnp.int32), pad_iota % N])
    dst_p = jnp.concatenate(
        [edge_index[1].astype(jnp.int32), (pad_iota * 37) % N])
    attr2_p = jnp.concatenate(
        [edge_attr, jnp.zeros((npad, D_EDGE), jnp.float32)]).reshape(
            EPAD // 2, 2 * D_EDGE)
    xcols = jnp.array([32 * (w // 16) + w % 16 for w in range(HIDDEN // 2)],
                      jnp.int32)
    wx, bx = edge_W[:, :, xcols], edge_b[:, xcols]
    wy, by = edge_W[:, :, xcols + 16], edge_b[:, xcols + 16]

    h = _encoder(x, node_depth, type_table, attr_table, depth_table)
    for i in range(LAYERS):
        ei = _edge_encode(attr2_p, wx[i], bx[i], wy[i], by[i])
        agg = _sc_layer(h, ei, src_p, dst_p)
        h = _mlp(1.0 + eps[i], h, agg, W1[i], b1[i], W2[i], b2[i])
    sums, counts = _pool(batch, h)
    h_graph = sums / jnp.maximum(counts.reshape(NUM_GRAPHS, 1), 1.0)
    return _heads(h_graph, pred_W, pred_b)
